# baseline (speedup 1.0000x reference)
import jax
import jax.numpy as jnp
from jax import lax
from jax.experimental import pallas as pl
from jax.experimental.pallas import tpu as pltpu
from jax.experimental.pallas import tpu_sc as plsc

_N_USERS = 50000
_N_ITEMS = 50000
_N = _N_USERS + _N_ITEMS
_E = 1600000
_NS = 16

_NP = 100096
_RPT = _NP // _NS
_ZB = 184

_CHUNK = 1024
_G = 128
_GP = _CHUNK // _G
_CR = _CHUNK // 128
_EPT = 100352
_NCHUNK = _EPT // _CHUNK
_EPAD = _EPT * _NS
_ERB = _EPAD // 128


def _add_body(a_ref, b_ref, o_ref):
    o_ref[...] = a_ref[...] + b_ref[...]


def _mean4_body(e0_ref, e1_ref, e2_ref, e3_ref, o_ref):
    o_ref[...] = ((e0_ref[...] + e1_ref[...])
                  + (e2_ref[...] + e3_ref[...])) * 0.25


def _sc_body(colr, rowr, valr, e0,
             e1, e2, e3,
             acc, gsem):
    f32 = jnp.float32
    i32 = jnp.int32
    pl.run_scoped(
        lambda colbuf, rowbuf, valbuf, rows, zbuf, wtmp: _sc_inner(
            colr, rowr, valr, e0, e1, e2, e3, acc, gsem,
            colbuf, rowbuf, valbuf, rows, zbuf, wtmp),
        pltpu.VMEM((_CR, 128), i32),
        pltpu.VMEM((_CR, 128), i32),
        pltpu.VMEM((_CR, 128), f32),
        pltpu.VMEM((_CHUNK, 16), f32),
        pltpu.VMEM((_ZB, 16), f32),
        pltpu.VMEM((_ZB, 16), f32),
    )


def _sc_inner(colr, rowr, valr, e0, e1, e2, e3, acc, gsem,
              colbuf, rowbuf, valbuf, rows, zbuf, wtmp):
    c = lax.axis_index("c")
    s = lax.axis_index("s")
    half = c * _NP
    halfv = lax.broadcast(half, (16,))

    zvec = jnp.zeros((16,), jnp.float32)

    def zb_body(i, _):
        zbuf[i] = zvec
        return 0

    lax.fori_loop(0, _ZB, zb_body, 0)
    for k in range(_RPT // _ZB):
        pltpu.sync_copy(zbuf, acc.at[pl.ds(s * _RPT + k * _ZB, _ZB)])
    plsc.subcore_barrier()

    ebase = s * (_EPT // 128)
    for src, dst in ((e0, e1), (e1, e2), (e2, e3)):

        def chunk_body(ch, _, src=src):
            r0 = ebase + ch * _CR
            pltpu.sync_copy(colr.at[pl.ds(r0, _CR)], colbuf)
            pltpu.sync_copy(rowr.at[pl.ds(r0, _CR)], rowbuf)
            pltpu.sync_copy(valr.at[pl.ds(r0, _CR)], valbuf)

            def off_body(i, _):
                jj = i // 8
                tt = i - jj * 8
                colbuf[jj, pl.ds(tt * 16, 16)] = (
                    colbuf[jj, pl.ds(tt * 16, 16)] + halfv)
                return 0

            lax.fori_loop(0, _CHUNK // 16, off_body, 0)

            cps = [pltpu.async_copy(src.at[colbuf.at[j]],
                                    rows.at[pl.ds(j * _G, _G)], gsem)
                   for j in range(_GP)]
            for cp in cps:
                cp.wait()

            def scale_body(g, _):
                jj = g // 8
                tt = g - jj * 8
                vv = valbuf[jj, pl.ds(tt * 16, 16)]
                base = g * 16
                for e in range(16):
                    sv = lax.broadcast(vv[e], (16,))
                    rows[base + e] = rows[base + e] * sv
                return 0

            lax.fori_loop(0, _CHUNK // 16, scale_body, 0)
            for j in range(_GP):
                pltpu.sync_copy(rows.at[pl.ds(j * _G, _G)],
                                acc.at[rowbuf.at[j]], add=True)
            return 0

        lax.fori_loop(0, _NCHUNK, chunk_body, 0)
        plsc.subcore_barrier()

        def wb_body(k, _, dst=dst):
            pltpu.sync_copy(acc.at[pl.ds(s * _RPT + k * _ZB, _ZB)], wtmp)
            pltpu.sync_copy(wtmp, dst.at[pl.ds(half + s * _RPT + k * _ZB, _ZB)])
            pltpu.sync_copy(zbuf, acc.at[pl.ds(s * _RPT + k * _ZB, _ZB)])
            return 0

        lax.fori_loop(0, _RPT // _ZB, wb_body, 0)
        plsc.subcore_barrier()


def kernel(adj_index, adj_values, user_emb, user_emb_pre, item_emb, item_emb_pre):
    f32 = jnp.float32
    pad = _EPAD - _E
    colr = jnp.pad(adj_index[1], (0, pad)).reshape(_ERB, 128)
    rowr = jnp.pad(adj_index[0], (0, pad)).reshape(_ERB, 128)
    valr = jnp.pad(adj_values, (0, pad)).reshape(_ERB, 128)

    nr = _N * 32 // 128
    allemb = jnp.concatenate([user_emb, item_emb], axis=0).reshape(nr, 128)
    allpre = jnp.concatenate([user_emb_pre, item_emb_pre], axis=0).reshape(nr, 128)
    blk = 1000
    ego = pl.pallas_call(
        _add_body,
        grid=(nr // blk,),
        in_specs=[pl.BlockSpec((blk, 128), lambda r: (r, 0)),
                  pl.BlockSpec((blk, 128), lambda r: (r, 0))],
        out_specs=pl.BlockSpec((blk, 128), lambda r: (r, 0)),
        out_shape=jax.ShapeDtypeStruct((nr, 128), f32),
    )(allemb, allpre)
    e0 = jnp.pad(ego.reshape(_N, 2, 16).transpose(1, 0, 2),
                 ((0, 0), (0, _NP - _N), (0, 0)))

    mesh = plsc.VectorSubcoreMesh(core_axis_name="c", subcore_axis_name="s")
    e1, e2, e3 = pl.kernel(
        _sc_body,
        out_type=[
            jax.ShapeDtypeStruct((2 * _NP, 16), f32),
            jax.ShapeDtypeStruct((2 * _NP, 16), f32),
            jax.ShapeDtypeStruct((2 * _NP, 16), f32),
        ],
        mesh=mesh,
        compiler_params=pltpu.CompilerParams(use_tc_tiling_on_sc=False),
        scratch_types=[
            pltpu.VMEM_SHARED((_NP, 16), f32),
            pltpu.SemaphoreType.DMA,
        ],
    )(colr, rowr, valr, e0.reshape(2 * _NP, 16))

    mr = 2 * _NP * 16 // 128
    mblk = 3128
    es = [e0.reshape(mr, 128)] + [e.reshape(mr, 128) for e in (e1, e2, e3)]
    s4 = pl.pallas_call(
        _mean4_body,
        grid=(mr // mblk,),
        in_specs=[pl.BlockSpec((mblk, 128), lambda r: (r, 0))
                  for _ in range(4)],
        out_specs=pl.BlockSpec((mblk, 128), lambda r: (r, 0)),
        out_shape=jax.ShapeDtypeStruct((mr, 128), f32),
    )(*es)
    mean = s4.reshape(2, _NP, 16)[:, :_N].transpose(1, 0, 2).reshape(_N, 32)
    return (mean[:_N_USERS], mean[_N_USERS:])
